# baseline (speedup 1.0000x reference)
import jax
import jax.numpy as jnp
from jax import lax
from jax.experimental import pallas as pl
from jax.experimental.pallas import tpu as pltpu
from jax.experimental.pallas import tpu_sc as plsc

BATCH = 16384
DIM = 64
LINES = 500000
LINE_W = 128
NC = 2
NS = 16
NW = NC * NS
BPW = BATCH // NW
L = 16
CH = 128
NCHUNK = BPW // CH


def _body(tw_hbm, cw_hbm, temb_hbm, cemb_hbm, out_hbm,
          idx_t, idx_c, line_t, line_c,
          rt0, rt1, rc0, rc1, out_v,
          st0, st1, sc0, sc1):
    cid = lax.axis_index("c")
    sid = lax.axis_index("s")
    wid = sid * NC + cid
    base = wid * BPW

    for j in range(NCHUNK):
        pltpu.sync_copy(tw_hbm.at[pl.ds(base + j * CH, CH)], idx_t.at[j])
        pltpu.sync_copy(cw_hbm.at[pl.ds(base + j * CH, CH)], idx_c.at[j])
    for j in range(NCHUNK):
        for k in range(CH // L):
            line_t[j, pl.ds(k * L, L)] = idx_t[j, pl.ds(k * L, L)] >> 1
            line_c[j, pl.ds(k * L, L)] = idx_c[j, pl.ds(k * L, L)] >> 1

    def copy_t(ch, buf, sem):
        return pltpu.make_async_copy(temb_hbm.at[line_t.at[ch]], buf, sem)

    def copy_c(ch, buf, sem):
        return pltpu.make_async_copy(cemb_hbm.at[line_c.at[ch]], buf, sem)

    copy_t(0, rt0, st0).start()
    copy_c(0, rc0, sc0).start()
    copy_t(1, rt1, st1).start()
    copy_c(1, rc1, sc1).start()

    lane = lax.iota(jnp.int32, L)

    def compute(ch, rt, rc):
        for g in range(CH // L):
            lidx = g * L + lane
            half_t = (idx_t[ch, pl.ds(g * L, L)] & 1) * DIM
            half_c = (idx_c[ch, pl.ds(g * L, L)] & 1) * DIM
            acc = jnp.zeros((L,), jnp.float32)
            for d in range(DIM):
                tv = plsc.load_gather(rt, [lidx, half_t + d])
                cv = plsc.load_gather(rc, [lidx, half_c + d])
                acc = acc + tv * cv
            out_v[pl.ds(ch * CH + g * L, L)] = acc

    def step(s, carry):
        ch0 = 2 * s
        ch1 = 2 * s + 1
        copy_t(ch0, rt0, st0).wait()
        copy_c(ch0, rc0, sc0).wait()
        compute(ch0, rt0, rc0)
        @pl.when(ch0 + 2 < NCHUNK)
        def _():
            copy_t(ch0 + 2, rt0, st0).start()
            copy_c(ch0 + 2, rc0, sc0).start()
        copy_t(ch1, rt1, st1).wait()
        copy_c(ch1, rc1, sc1).wait()
        compute(ch1, rt1, rc1)
        @pl.when(ch1 + 2 < NCHUNK)
        def _():
            copy_t(ch1 + 2, rt1, st1).start()
            copy_c(ch1 + 2, rc1, sc1).start()
        return carry

    lax.fori_loop(0, NCHUNK // 2, step, 0)

    pltpu.sync_copy(out_v, out_hbm.at[pl.ds(base, BPW)])


def kernel(target_word, context_word, target_emb, context_emb):
    tw = target_word.astype(jnp.int32)
    cw = context_word.astype(jnp.int32)
    one = jnp.where(tw[0] < 0, jnp.float32(2.0), jnp.float32(1.0))
    temb_d = target_emb.reshape(LINES, LINE_W) * one
    cemb_d = context_emb.reshape(LINES, LINE_W) * one
    mesh = plsc.VectorSubcoreMesh(
        core_axis_name="c", subcore_axis_name="s",
        num_cores=NC, num_subcores=NS)
    run = pl.kernel(
        _body,
        out_type=jax.ShapeDtypeStruct((BATCH,), jnp.float32),
        mesh=mesh,
        scratch_types=[
            pltpu.VMEM((NCHUNK, CH), jnp.int32),
            pltpu.VMEM((NCHUNK, CH), jnp.int32),
            pltpu.VMEM((NCHUNK, CH), jnp.int32),
            pltpu.VMEM((NCHUNK, CH), jnp.int32),
            pltpu.VMEM((CH, LINE_W), jnp.float32),
            pltpu.VMEM((CH, LINE_W), jnp.float32),
            pltpu.VMEM((CH, LINE_W), jnp.float32),
            pltpu.VMEM((CH, LINE_W), jnp.float32),
            pltpu.VMEM((BPW,), jnp.float32),
            pltpu.SemaphoreType.DMA,
            pltpu.SemaphoreType.DMA,
            pltpu.SemaphoreType.DMA,
            pltpu.SemaphoreType.DMA,
        ],
        compiler_params=pltpu.CompilerParams(
            needs_layout_passes=False, use_tc_tiling_on_sc=True),
    )
    return run(tw, cw, temb_d, cemb_d)


# speedup vs baseline: 1.5902x; 1.5902x over previous
import jax
import jax.numpy as jnp
from jax import lax
from jax.experimental import pallas as pl
from jax.experimental.pallas import tpu as pltpu
from jax.experimental.pallas import tpu_sc as plsc

BATCH = 16384
DIM = 64
LINES = 500000
LINE_W = 128
NC = 2
NS = 16
NW = NC * NS
BPW = BATCH // NW
L = 16
CH = 128
NCHUNK = BPW // CH


def _body(tw_hbm, cw_hbm, temb_hbm, cemb_hbm, out_hbm,
          idx_t, idx_c, line_t, line_c,
          rt0, rt1, rc0, rc1, out_v,
          st0, st1, sc0, sc1):
    cid = lax.axis_index("c")
    sid = lax.axis_index("s")
    wid = sid * NC + cid
    base = wid * BPW

    for j in range(NCHUNK):
        pltpu.sync_copy(tw_hbm.at[pl.ds(base + j * CH, CH)], idx_t.at[j])
        pltpu.sync_copy(cw_hbm.at[pl.ds(base + j * CH, CH)], idx_c.at[j])
    for j in range(NCHUNK):
        for k in range(CH // L):
            line_t[j, pl.ds(k * L, L)] = idx_t[j, pl.ds(k * L, L)] >> 1
            line_c[j, pl.ds(k * L, L)] = idx_c[j, pl.ds(k * L, L)] >> 1

    def copy_t(ch, buf, sem):
        return pltpu.make_async_copy(temb_hbm.at[line_t.at[ch]], buf, sem)

    def copy_c(ch, buf, sem):
        return pltpu.make_async_copy(cemb_hbm.at[line_c.at[ch]], buf, sem)

    copy_t(0, rt0, st0).start()
    copy_c(0, rc0, sc0).start()
    copy_t(1, rt1, st1).start()
    copy_c(1, rc1, sc1).start()

    lane = lax.iota(jnp.int32, L)

    def compute(ch, rt, rc):
        for g in range(CH // L):
            lidx = g * L + lane
            half_t = (idx_t[ch, pl.ds(g * L, L)] & 1) * DIM
            half_c = (idx_c[ch, pl.ds(g * L, L)] & 1) * DIM
            acc = jnp.zeros((L,), jnp.float32)
            for d in range(DIM):
                tv = plsc.load_gather(rt, [lidx, half_t + d])
                cv = plsc.load_gather(rc, [lidx, half_c + d])
                acc = acc + tv * cv
            out_v[pl.ds(ch * CH + g * L, L)] = acc

    def step(s, carry):
        ch0 = 2 * s
        ch1 = 2 * s + 1
        copy_t(ch0, rt0, st0).wait()
        copy_c(ch0, rc0, sc0).wait()
        compute(ch0, rt0, rc0)
        @pl.when(ch0 + 2 < NCHUNK)
        def _():
            copy_t(ch0 + 2, rt0, st0).start()
            copy_c(ch0 + 2, rc0, sc0).start()
        copy_t(ch1, rt1, st1).wait()
        copy_c(ch1, rc1, sc1).wait()
        compute(ch1, rt1, rc1)
        @pl.when(ch1 + 2 < NCHUNK)
        def _():
            copy_t(ch1 + 2, rt1, st1).start()
            copy_c(ch1 + 2, rc1, sc1).start()
        return carry

    lax.fori_loop(0, NCHUNK // 2, step, 0)

    pltpu.sync_copy(out_v, out_hbm.at[pl.ds(base, BPW)])


def kernel(target_word, context_word, target_emb, context_emb):
    tw = target_word.astype(jnp.int32)
    cw = context_word.astype(jnp.int32)
    temb_d = target_emb.reshape(LINES, LINE_W)
    cemb_d = context_emb.reshape(LINES, LINE_W)
    mesh = plsc.VectorSubcoreMesh(
        core_axis_name="c", subcore_axis_name="s",
        num_cores=NC, num_subcores=NS)
    run = pl.kernel(
        _body,
        out_type=jax.ShapeDtypeStruct((BATCH,), jnp.float32),
        mesh=mesh,
        scratch_types=[
            pltpu.VMEM((NCHUNK, CH), jnp.int32),
            pltpu.VMEM((NCHUNK, CH), jnp.int32),
            pltpu.VMEM((NCHUNK, CH), jnp.int32),
            pltpu.VMEM((NCHUNK, CH), jnp.int32),
            pltpu.VMEM((CH, LINE_W), jnp.float32),
            pltpu.VMEM((CH, LINE_W), jnp.float32),
            pltpu.VMEM((CH, LINE_W), jnp.float32),
            pltpu.VMEM((CH, LINE_W), jnp.float32),
            pltpu.VMEM((BPW,), jnp.float32),
            pltpu.SemaphoreType.DMA,
            pltpu.SemaphoreType.DMA,
            pltpu.SemaphoreType.DMA,
            pltpu.SemaphoreType.DMA,
        ],
        compiler_params=pltpu.CompilerParams(
            needs_layout_passes=False, use_tc_tiling_on_sc=True),
    )
    return run(tw, cw, temb_d, cemb_d)
